# speedup vs baseline: 5.4907x; 5.4907x over previous
import functools

import jax
import jax.numpy as jnp
from jax import lax
from jax.experimental import pallas as pl
from jax.experimental.pallas import tpu as pltpu
from jax.experimental.pallas import tpu_sc as plsc

NUM_CORES = 2
NUM_SUBCORES = 16
CHUNK = 128
ZCHUNK = 400


def _sc_agg(h, ei3, zeros):
    n, d = h.shape
    n_chunks = ei3.shape[1]
    chunks_per_core = n_chunks // NUM_CORES
    n_zchunks = n // ZCHUNK

    @functools.partial(
        pl.kernel,
        out_type=jax.ShapeDtypeStruct((NUM_CORES, n, d), jnp.float32),
        mesh=plsc.VectorSubcoreMesh(core_axis_name="c", subcore_axis_name="s"),
        scratch_types=[
            pltpu.VMEM((CHUNK,), jnp.int32),
            pltpu.VMEM((CHUNK,), jnp.int32),
            pltpu.VMEM((CHUNK, d), jnp.float32),
            pltpu.VMEM_SHARED((n, d), jnp.float32),
        ],
    )
    def k(h_hbm, ei_hbm, z_hbm, out_hbm, sidx, didx, rows, acc):
        cid = lax.axis_index("c")
        sid = lax.axis_index("s")

        @pl.loop(sid, n_zchunks, step=NUM_SUBCORES)
        def _(z):
            pltpu.sync_copy(z_hbm, acc.at[pl.ds(z * ZCHUNK, ZCHUNK), :])

        plsc.subcore_barrier()

        @pl.loop(cid * chunks_per_core + sid,
                 (cid + 1) * chunks_per_core,
                 step=NUM_SUBCORES)
        def _(chunk):
            pltpu.sync_copy(ei_hbm.at[0, chunk, :], sidx)
            pltpu.sync_copy(ei_hbm.at[1, chunk, :], didx)
            pltpu.sync_copy(h_hbm.at[sidx], rows)
            pltpu.sync_copy(rows, acc.at[didx], add=True)

        plsc.subcore_barrier()

        @pl.loop(sid, n_zchunks, step=NUM_SUBCORES)
        def _(z):
            pltpu.sync_copy(acc.at[pl.ds(z * ZCHUNK, ZCHUNK), :],
                            out_hbm.at[cid, pl.ds(z * ZCHUNK, ZCHUNK), :])

    return k(h, ei3, zeros)


def _mlp(x, p, W1, b1, W2, b2, relu_out, block):
    n, d = x.shape

    def body(x_ref, p0_ref, p1_ref, w1_ref, b1_ref, w2_ref, b2_ref, o_ref):
        h = x_ref[...] + p0_ref[...] + p1_ref[...]
        t = jnp.dot(h, w1_ref[...], preferred_element_type=jnp.float32)
        t = jnp.maximum(t + b1_ref[...], 0.0)
        o = jnp.dot(t, w2_ref[...], preferred_element_type=jnp.float32)
        o = o + b2_ref[...]
        if relu_out:
            o = jnp.maximum(o, 0.0)
        o_ref[...] = o

    row_spec = pl.BlockSpec((block, d), lambda i: (i, 0))
    full_mat = pl.BlockSpec((d, d), lambda i: (0, 0))
    full_vec = pl.BlockSpec((1, d), lambda i: (0, 0))
    return pl.pallas_call(
        body,
        grid=(n // block,),
        in_specs=[row_spec, row_spec, row_spec,
                  full_mat, full_vec, full_mat, full_vec],
        out_specs=row_spec,
        out_shape=jax.ShapeDtypeStruct((n, d), jnp.float32),
    )(x, p[0], p[1], W1, b1.reshape(1, d), W2, b2.reshape(1, d))


def kernel(x, edge_index,
           W1_0, b1_0, W2_0, b2_0,
           W1_1, b1_1, W2_1, b2_1,
           W1_2, b1_2, W2_2, b2_2):
    n, d = x.shape
    e = edge_index.shape[1]
    ei3 = edge_index.reshape(2, e // CHUNK, CHUNK)
    zeros = jnp.zeros((ZCHUNK, d), jnp.float32)

    h = x
    for i, (W1, b1, W2, b2) in enumerate([
            (W1_0, b1_0, W2_0, b2_0),
            (W1_1, b1_1, W2_1, b2_1),
            (W1_2, b1_2, W2_2, b2_2)]):
        p = _sc_agg(h, ei3, zeros)
        h = _mlp(h, p, W1, b1, W2, b2, relu_out=(i < 2), block=1000)
    return h
